# baseline (speedup 1.0000x reference)
import functools

import jax
import jax.numpy as jnp
from jax import lax
from jax.experimental import pallas as pl
from jax.experimental.pallas import tpu as pltpu
from jax.experimental.pallas import tpu_sc as plsc

N = 200000
S = 5000
NS = 16
L = 16

CH = 12544
OV = NS * CH - N
OV16 = OV // L
SP = 5120
AP = 320
ZB = SP // NS

_mesh = plsc.VectorSubcoreMesh(core_axis_name="c", subcore_axis_name="s")


@functools.partial(
    pl.kernel,
    mesh=_mesh,
    compiler_params=pltpu.CompilerParams(needs_layout_passes=False,
                                         use_tc_tiling_on_sc=False),
    out_type=jax.ShapeDtypeStruct((4, N), jnp.float32),
    scratch_types=[
        pltpu.VMEM((CH,), jnp.int32),
        pltpu.VMEM((CH,), jnp.float32),
        pltpu.VMEM((CH,), jnp.float32),
        pltpu.VMEM((CH,), jnp.float32),
        pltpu.VMEM((CH,), jnp.float32),
        pltpu.VMEM((SP,), jnp.float32),
        pltpu.VMEM((SP,), jnp.float32),
        pltpu.VMEM((AP,), jnp.int32),
        pltpu.VMEM((AP,), jnp.float32),
        pltpu.VMEM_SHARED((SP,), jnp.float32),
        pltpu.VMEM_SHARED((SP,), jnp.float32),
        pltpu.SemaphoreType.DMA,
    ],
)
def _sc_kernel(packed_h, ah_h, out_h,
               sid_v, e_v, vals_v, raw_v, cor_v, sums_v, corr_v,
               aidx_v, acorr_v, sums_sh, corr_sh, sem):
    c = lax.axis_index("c")
    s = lax.axis_index("s")
    is_last = s == NS - 1
    base = jnp.where(is_last, N - CH, s * CH)
    abase = jnp.where(is_last, S - AP, s * AP)

    pltpu.sync_copy(ah_h.at[pl.ds(c * S + abase, AP)], aidx_v)
    agather = pltpu.async_copy(packed_h.at[3].at[aidx_v], acorr_v, sem)

    pltpu.sync_copy(packed_h.at[0, pl.ds(base, CH)], raw_v)
    pltpu.sync_copy(packed_h.at[1, pl.ds(base, CH)], cor_v)
    pltpu.sync_copy(packed_h.at[2, pl.ds(base, CH)], e_v)

    def _zbody(j, carry):
        sums_v[pl.ds(j * L, L)] = jnp.zeros((L,), jnp.float32)
        return carry
    lax.fori_loop(0, ZB // L, _zbody, 0)
    pltpu.sync_copy(sums_v.at[pl.ds(0, ZB)], sums_sh.at[pl.ds(s * ZB, ZB)])

    agather.wait()
    pltpu.sync_copy(acorr_v, corr_sh.at[pl.ds(abase, AP)])

    jmin = jnp.where(is_last, OV16, 0)

    @plsc.parallel_loop(0, CH // L, unroll=8)
    def _vbody(j):
        sid16 = plsc.bitcast(raw_v[pl.ds(j * L, L)], jnp.int32)
        id16 = plsc.bitcast(cor_v[pl.ds(j * L, L)], jnp.int32)
        e16 = e_v[pl.ds(j * L, L)]
        sid_v[pl.ds(j * L, L)] = sid16
        keep = jnp.logical_and(id16 == c, j >= jmin)
        vals_v[pl.ds(j * L, L)] = jnp.where(keep, e16, jnp.zeros((L,), jnp.float32))

    plsc.subcore_barrier()

    pltpu.sync_copy(vals_v, sums_sh.at[sid_v], add=True)

    plsc.subcore_barrier()

    pltpu.sync_copy(sums_sh, sums_v)
    pltpu.sync_copy(corr_sh, corr_v)

    @plsc.parallel_loop(0, CH // L, unroll=8)
    def _gbody(j):
        sid16 = sid_v[pl.ds(j * L, L)]
        raw = plsc.load_gather(sums_v, [sid16])
        cfac = plsc.load_gather(corr_v, [sid16])
        raw_v[pl.ds(j * L, L)] = raw
        cor_v[pl.ds(j * L, L)] = raw * cfac

    pltpu.sync_copy(raw_v, out_h.at[2 * c, pl.ds(base, CH)])
    pltpu.sync_copy(cor_v, out_h.at[2 * c + 1, pl.ds(base, CH)])


def kernel(pred_sid, pred_energy_corr_factor, pred_beta, recHitEnergy,
           recHitID, alpha_idx_tracks, alpha_idx_hits):
    del pred_beta
    bc = lambda a: jax.lax.bitcast_convert_type(a, jnp.float32)
    packed = jnp.concatenate(
        [bc(pred_sid), bc(recHitID), recHitEnergy, pred_energy_corr_factor],
        axis=1).T
    alpha_all = jnp.concatenate([alpha_idx_hits.astype(jnp.int32),
                                 alpha_idx_tracks.astype(jnp.int32)])
    out4 = _sc_kernel(packed, alpha_all).reshape(4, N, 1)
    return (out4[2], out4[3], out4[0], out4[1])


# speedup vs baseline: 5.2312x; 5.2312x over previous
import functools

import jax
import jax.numpy as jnp
from jax import lax
from jax.experimental import pallas as pl
from jax.experimental.pallas import tpu as pltpu
from jax.experimental.pallas import tpu_sc as plsc

N = 200000
S = 5000
NS = 16
L = 16

CH = 12544
OV = NS * CH - N
OV16 = OV // L
SP = 5120
AP = 320
ZB = SP // NS

_mesh = plsc.VectorSubcoreMesh(core_axis_name="c", subcore_axis_name="s")


@functools.partial(
    pl.kernel,
    mesh=_mesh,
    compiler_params=pltpu.CompilerParams(needs_layout_passes=False,
                                         use_tc_tiling_on_sc=False),
    out_type=jax.ShapeDtypeStruct((4 * N,), jnp.float32),
    scratch_types=[
        pltpu.VMEM((CH,), jnp.int32),
        pltpu.VMEM((CH,), jnp.int32),
        pltpu.VMEM((CH,), jnp.int32),
        pltpu.VMEM((CH,), jnp.float32),
        pltpu.VMEM((CH,), jnp.float32),
        pltpu.VMEM((CH,), jnp.float32),
        pltpu.VMEM((SP,), jnp.float32),
        pltpu.VMEM((SP,), jnp.int32),
        pltpu.VMEM((AP,), jnp.int32),
        pltpu.VMEM((AP,), jnp.int32),
        pltpu.VMEM_SHARED((SP,), jnp.float32),
        pltpu.VMEM_SHARED((SP,), jnp.int32),
        pltpu.SemaphoreType.DMA,
    ],
)
def _sc_kernel(packed_h, ah_h, out_h,
               sid_v, id_v, eb_v, vals_v, raw_v, cor_v, sums_v, corr_v,
               aidx_v, acorr_v, sums_sh, corr_sh, sem):
    c = lax.axis_index("c")
    s = lax.axis_index("s")
    is_last = s == NS - 1
    base = jnp.where(is_last, N - CH, s * CH)
    abase = jnp.where(is_last, S - AP, s * AP)

    pltpu.sync_copy(ah_h.at[pl.ds(c * S + abase, AP)], aidx_v)
    agather = pltpu.async_copy(packed_h.at[aidx_v], acorr_v, sem)

    pltpu.sync_copy(packed_h.at[pl.ds(base, CH)], sid_v)
    pltpu.sync_copy(packed_h.at[pl.ds(N + base, CH)], id_v)
    pltpu.sync_copy(packed_h.at[pl.ds(2 * N + base, CH)], eb_v)

    def _zbody(j, carry):
        sums_v[pl.ds(j * L, L)] = jnp.zeros((L,), jnp.float32)
        return carry
    lax.fori_loop(0, ZB // L, _zbody, 0)
    pltpu.sync_copy(sums_v.at[pl.ds(0, ZB)], sums_sh.at[pl.ds(s * ZB, ZB)])

    agather.wait()
    pltpu.sync_copy(acorr_v, corr_sh.at[pl.ds(abase, AP)])

    jmin = jnp.where(is_last, OV16, 0)

    @plsc.parallel_loop(0, CH // L, unroll=8)
    def _vbody(j):
        id16 = id_v[pl.ds(j * L, L)]
        e16 = plsc.bitcast(eb_v[pl.ds(j * L, L)], jnp.float32)
        keep = jnp.logical_and(id16 == c, j >= jmin)
        vals_v[pl.ds(j * L, L)] = jnp.where(keep, e16, jnp.zeros((L,), jnp.float32))

    plsc.subcore_barrier()

    pltpu.sync_copy(vals_v, sums_sh.at[sid_v], add=True)

    plsc.subcore_barrier()

    pltpu.sync_copy(sums_sh, sums_v)
    pltpu.sync_copy(corr_sh, corr_v)

    @plsc.parallel_loop(0, CH // L, unroll=8)
    def _gbody(j):
        sid16 = sid_v[pl.ds(j * L, L)]
        raw = plsc.load_gather(sums_v, [sid16])
        cfac = plsc.bitcast(plsc.load_gather(corr_v, [sid16]), jnp.float32)
        raw_v[pl.ds(j * L, L)] = raw
        cor_v[pl.ds(j * L, L)] = raw * cfac

    obase = 2 * c * N + base
    pltpu.sync_copy(raw_v, out_h.at[pl.ds(obase, CH)])
    pltpu.sync_copy(cor_v, out_h.at[pl.ds(obase + N, CH)])


def kernel(pred_sid, pred_energy_corr_factor, pred_beta, recHitEnergy,
           recHitID, alpha_idx_tracks, alpha_idx_hits):
    del pred_beta
    bc = lambda a: jax.lax.bitcast_convert_type(a, jnp.int32)
    packed = jnp.concatenate(
        [pred_sid, recHitID, bc(recHitEnergy), bc(pred_energy_corr_factor)],
        axis=1).T.reshape(4 * N)
    alpha_all = jnp.concatenate([alpha_idx_hits.astype(jnp.int32),
                                 alpha_idx_tracks.astype(jnp.int32)]) + 3 * N
    out4 = _sc_kernel(packed, alpha_all).reshape(4, N, 1)
    return (out4[2], out4[3], out4[0], out4[1])


# speedup vs baseline: 5.3606x; 1.0247x over previous
import functools

import jax
import jax.numpy as jnp
from jax import lax
from jax.experimental import pallas as pl
from jax.experimental.pallas import tpu as pltpu
from jax.experimental.pallas import tpu_sc as plsc

N = 200000
S = 5000
NS = 16
L = 16

CH = 12544
OV = NS * CH - N
OV16 = OV // L
SP = 5120
AP = 320
ZB = SP // NS

_mesh = plsc.VectorSubcoreMesh(core_axis_name="c", subcore_axis_name="s")


@functools.partial(
    pl.kernel,
    mesh=_mesh,
    compiler_params=pltpu.CompilerParams(needs_layout_passes=False,
                                         use_tc_tiling_on_sc=False),
    out_type=jax.ShapeDtypeStruct((4 * N,), jnp.float32),
    scratch_types=[
        pltpu.VMEM((CH,), jnp.int32),
        pltpu.VMEM((CH,), jnp.int32),
        pltpu.VMEM((CH,), jnp.int32),
        pltpu.VMEM((CH,), jnp.float32),
        pltpu.VMEM((CH,), jnp.float32),
        pltpu.VMEM((CH,), jnp.float32),
        pltpu.VMEM((SP,), jnp.float32),
        pltpu.VMEM((SP,), jnp.int32),
        pltpu.VMEM((AP,), jnp.int32),
        pltpu.VMEM((AP,), jnp.int32),
        pltpu.VMEM_SHARED((SP,), jnp.float32),
        pltpu.VMEM_SHARED((SP,), jnp.int32),
        pltpu.SemaphoreType.DMA,
    ],
)
def _sc_kernel(packed_h, ah_h, out_h,
               sid_v, id_v, eb_v, vals_v, raw_v, cor_v, sums_v, corr_v,
               aidx_v, acorr_v, sums_sh, corr_sh, sem):
    c = lax.axis_index("c")
    s = lax.axis_index("s")
    is_last = s == NS - 1
    base = jnp.where(is_last, N - CH, s * CH)
    abase = jnp.where(is_last, S - AP, s * AP)

    pltpu.sync_copy(ah_h.at[pl.ds(c * S + abase, AP)], aidx_v)
    agather = pltpu.async_copy(packed_h.at[aidx_v], acorr_v, sem)

    cp_sid = pltpu.async_copy(packed_h.at[pl.ds(base, CH)], sid_v, sem)
    cp_id = pltpu.async_copy(packed_h.at[pl.ds(N + base, CH)], id_v, sem)
    cp_eb = pltpu.async_copy(packed_h.at[pl.ds(2 * N + base, CH)], eb_v, sem)

    def _zbody(j, carry):
        sums_v[pl.ds(j * L, L)] = jnp.zeros((L,), jnp.float32)
        return carry
    lax.fori_loop(0, ZB // L, _zbody, 0)
    pltpu.sync_copy(sums_v.at[pl.ds(0, ZB)], sums_sh.at[pl.ds(s * ZB, ZB)])

    agather.wait()
    pltpu.sync_copy(acorr_v, corr_sh.at[pl.ds(abase, AP)])
    cp_sid.wait()
    cp_id.wait()
    cp_eb.wait()

    jmin = jnp.where(is_last, OV16, 0)

    @plsc.parallel_loop(0, CH // L, unroll=8)
    def _vbody(j):
        id16 = id_v[pl.ds(j * L, L)]
        e16 = plsc.bitcast(eb_v[pl.ds(j * L, L)], jnp.float32)
        keep = jnp.logical_and(id16 == c, j >= jmin)
        vals_v[pl.ds(j * L, L)] = jnp.where(keep, e16, jnp.zeros((L,), jnp.float32))

    plsc.subcore_barrier()

    pltpu.sync_copy(vals_v, sums_sh.at[sid_v], add=True)

    plsc.subcore_barrier()

    pltpu.sync_copy(sums_sh, sums_v)
    pltpu.sync_copy(corr_sh, corr_v)

    @plsc.parallel_loop(0, CH // L, unroll=8)
    def _gbody(j):
        sid16 = sid_v[pl.ds(j * L, L)]
        raw = plsc.load_gather(sums_v, [sid16])
        cfac = plsc.bitcast(plsc.load_gather(corr_v, [sid16]), jnp.float32)
        raw_v[pl.ds(j * L, L)] = raw
        cor_v[pl.ds(j * L, L)] = raw * cfac

    obase = (1 - c) * 2 * N + base
    pltpu.sync_copy(raw_v, out_h.at[pl.ds(obase, CH)])
    pltpu.sync_copy(cor_v, out_h.at[pl.ds(obase + N, CH)])


def kernel(pred_sid, pred_energy_corr_factor, pred_beta, recHitEnergy,
           recHitID, alpha_idx_tracks, alpha_idx_hits):
    del pred_beta
    bc = lambda a: jax.lax.bitcast_convert_type(a, jnp.int32)
    packed = jnp.concatenate(
        [pred_sid, recHitID, bc(recHitEnergy), bc(pred_energy_corr_factor)],
        axis=1).T.reshape(4 * N)
    alpha_all = jnp.concatenate([alpha_idx_hits.astype(jnp.int32),
                                 alpha_idx_tracks.astype(jnp.int32)]) + 3 * N
    out4 = _sc_kernel(packed, alpha_all).reshape(4, N, 1)
    return (out4[0], out4[1], out4[2], out4[3])
